# baseline (speedup 1.0000x reference)
import jax
import jax.numpy as jnp
from jax.experimental import pallas as pl
from jax.experimental.pallas import tpu as pltpu


def _fused_kernel(x_ref, y_ref, wqt_ref, bq_ref, wkt_ref, bk_ref,
                  wvt_ref, bv_ref, o_ref, k_scr, v_scr):
    i = pl.program_id(1)

    @pl.when(i == 0)
    def _project_kv():
        yb = y_ref[0]
        k_scr[...] = jnp.dot(yb, wkt_ref[...],
                             preferred_element_type=jnp.float32) + bk_ref[...]
        vb = jnp.dot(yb.astype(jnp.bfloat16), wvt_ref[...].astype(jnp.bfloat16),
                     preferred_element_type=jnp.float32) + bv_ref[...]
        v_scr[...] = vb.astype(jnp.bfloat16)

    @pl.when(i > 0)
    def _attend():
        xb = x_ref[0]
        q = jnp.dot(xb, wqt_ref[...],
                    preferred_element_type=jnp.float32) + bq_ref[...]
        s = jax.lax.dot_general(q, k_scr[...], (((1,), (1,)), ((), ())),
                                preferred_element_type=jnp.float32)
        m = jnp.max(s, axis=-1, keepdims=True)
        p = jnp.exp(s - m)
        l = jnp.sum(p, axis=-1, keepdims=True)
        o = jnp.dot(p.astype(jnp.bfloat16), v_scr[...],
                    preferred_element_type=jnp.float32)
        o_ref[0] = o / l + xb


def kernel(x, y, Wq, bq, Wk, bk, Wv, bv):
    B, SX, D = x.shape
    SY = y.shape[1]
    bq_rows = min(512, SX)
    nq = SX // bq_rows

    wqt = Wq.T
    wkt = Wk.T
    wvt = Wv.T
    bq2 = bq.reshape(1, D)
    bk2 = bk.reshape(1, D)
    bv2 = bv.reshape(1, D)

    def qi(b, i):
        return (b, jnp.maximum(i - 1, 0), 0)

    out = pl.pallas_call(
        _fused_kernel,
        grid=(B, nq + 1),
        in_specs=[
            pl.BlockSpec((1, bq_rows, D), qi),
            pl.BlockSpec((1, SY, D), lambda b, i: (b, 0, 0)),
            pl.BlockSpec((D, D), lambda b, i: (0, 0)),
            pl.BlockSpec((1, D), lambda b, i: (0, 0)),
            pl.BlockSpec((D, D), lambda b, i: (0, 0)),
            pl.BlockSpec((1, D), lambda b, i: (0, 0)),
            pl.BlockSpec((D, D), lambda b, i: (0, 0)),
            pl.BlockSpec((1, D), lambda b, i: (0, 0)),
        ],
        out_specs=pl.BlockSpec((1, bq_rows, D), qi),
        out_shape=jax.ShapeDtypeStruct((B, SX, D), jnp.float32),
        scratch_shapes=[
            pltpu.VMEM((SY, D), jnp.float32),
            pltpu.VMEM((SY, D), jnp.bfloat16),
        ],
    )(x, y, wqt, bq2, wkt, bk2, wvt, bv2)
    return out


# speedup vs baseline: 1.0471x; 1.0471x over previous
import jax
import jax.numpy as jnp
from jax.experimental import pallas as pl
from jax.experimental.pallas import tpu as pltpu


def _fused_kernel(x_ref, y_ref, wqt_ref, bq_ref, wkt_ref, bk_ref,
                  wvt_ref, bv_ref, o_ref, k_scr, v_scr):
    i = pl.program_id(1)

    @pl.when(i == 0)
    def _project_kv():
        yb = y_ref[0]
        k_scr[...] = jnp.dot(yb, wkt_ref[...],
                             preferred_element_type=jnp.float32) + bk_ref[...]
        vb = jnp.dot(yb.astype(jnp.bfloat16), wvt_ref[...].astype(jnp.bfloat16),
                     preferred_element_type=jnp.float32) + bv_ref[...]
        v_scr[...] = vb.astype(jnp.bfloat16)

    @pl.when(i > 0)
    def _attend():
        xb = x_ref[0]
        rows = xb.shape[0]
        nsub = 2
        sub = rows // nsub
        for h in range(nsub):
            xh = xb[h * sub:(h + 1) * sub]
            q = jnp.dot(xh, wqt_ref[...],
                        preferred_element_type=jnp.float32) + bq_ref[...]
            s = jax.lax.dot_general(q, k_scr[...], (((1,), (1,)), ((), ())),
                                    preferred_element_type=jnp.float32)
            m = jnp.max(s, axis=-1, keepdims=True)
            p = jnp.exp(s - m)
            l = jnp.sum(p, axis=-1, keepdims=True)
            o = jnp.dot(p.astype(jnp.bfloat16), v_scr[...],
                        preferred_element_type=jnp.float32)
            o_ref[0, h * sub:(h + 1) * sub] = o / l + xh


def kernel(x, y, Wq, bq, Wk, bk, Wv, bv):
    B, SX, D = x.shape
    SY = y.shape[1]
    bq_rows = min(512, SX)
    nq = SX // bq_rows

    wqt = Wq.T
    wkt = Wk.T
    wvt = Wv.T
    bq2 = bq.reshape(1, D)
    bk2 = bk.reshape(1, D)
    bv2 = bv.reshape(1, D)

    def qi(b, i):
        return (b, jnp.maximum(i - 1, 0), 0)

    out = pl.pallas_call(
        _fused_kernel,
        grid=(B, nq + 1),
        in_specs=[
            pl.BlockSpec((1, bq_rows, D), qi),
            pl.BlockSpec((1, SY, D), lambda b, i: (b, 0, 0)),
            pl.BlockSpec((D, D), lambda b, i: (0, 0)),
            pl.BlockSpec((1, D), lambda b, i: (0, 0)),
            pl.BlockSpec((D, D), lambda b, i: (0, 0)),
            pl.BlockSpec((1, D), lambda b, i: (0, 0)),
            pl.BlockSpec((D, D), lambda b, i: (0, 0)),
            pl.BlockSpec((1, D), lambda b, i: (0, 0)),
        ],
        out_specs=pl.BlockSpec((1, bq_rows, D), qi),
        out_shape=jax.ShapeDtypeStruct((B, SX, D), jnp.float32),
        scratch_shapes=[
            pltpu.VMEM((SY, D), jnp.float32),
            pltpu.VMEM((SY, D), jnp.bfloat16),
        ],
    )(x, y, wqt, bq2, wkt, bk2, wvt, bv2)
    return out
